# baseline (speedup 1.0000x reference)
import functools

import jax
import jax.numpy as jnp
from jax import lax
from jax.experimental import pallas as pl
from jax.experimental.pallas import tpu as pltpu
from jax.experimental.pallas import tpu_sc as plsc

_D = 128
_SHARED = 32
_B = 16384
_NC = 2
_NS = 16
_NW = _NC * _NS
_BPW = _B // _NW
_CHUNK = 64
_NCHUNK = _BPW // _CHUNK


def _sc_body(x_hbm, tab_hbm, sh_hbm, out_hbm, idx_v, rows_v, sh_v, shblk_v, gsems, osem):
    wid = lax.axis_index("s") * _NC + lax.axis_index("c")
    base = wid * _BPW

    a_i0 = pltpu.async_copy(x_hbm.at[wid, 0], idx_v.at[0], gsems.at[0])
    a_ir = pltpu.async_copy(
        x_hbm.at[wid, pl.ds(1, _NCHUNK - 1)],
        idx_v.at[pl.ds(1, _NCHUNK - 1)],
        gsems.at[1],
    )
    a_sh = pltpu.async_copy(sh_hbm, sh_v, gsems.at[2])

    def gather(k):
        return pltpu.async_copy(
            tab_hbm.at[idx_v.at[k]],
            rows_v.at[pl.ds(k * _CHUNK, _CHUNK)],
            gsems.at[k],
        )

    a_i0.wait()
    gathers = [gather(0)]
    a_ir.wait()
    gathers += [gather(k) for k in range(1, _NCHUNK)]
    a_sh.wait()

    s0 = sh_v[pl.ds(0, 16)]
    s1 = sh_v[pl.ds(16, 16)]

    def blk_fill(i, carry):
        shblk_v[i, pl.ds(0, 16)] = s0
        shblk_v[i, pl.ds(16, 16)] = s1
        return carry

    lax.fori_loop(0, _BPW, blk_fill, 0)

    outs = [
        pltpu.async_copy(
            shblk_v,
            out_hbm.at[pl.ds(base, _BPW), pl.ds(0, _SHARED)],
            osem,
        )
    ]
    for k in range(_NCHUNK):
        gathers[k].wait()
        outs.append(
            pltpu.async_copy(
                rows_v.at[pl.ds(k * _CHUNK, _CHUNK), pl.ds(_SHARED, _D - _SHARED)],
                out_hbm.at[pl.ds(base + k * _CHUNK, _CHUNK), pl.ds(_SHARED, _D - _SHARED)],
                osem,
            )
        )
    for o in outs:
        o.wait()


_mesh = plsc.VectorSubcoreMesh(core_axis_name="c", subcore_axis_name="s")

_emb_lookup = pl.kernel(
    _sc_body,
    mesh=_mesh,
    compiler_params=pltpu.CompilerParams(use_tc_tiling_on_sc=False),
    out_type=jax.ShapeDtypeStruct((_B, _D), jnp.float32),
    scratch_types=[
        pltpu.VMEM((_NCHUNK, _CHUNK), jnp.int32),
        pltpu.VMEM((_BPW, _D), jnp.float32),
        pltpu.VMEM((_SHARED,), jnp.float32),
        pltpu.VMEM((_BPW, _SHARED), jnp.float32),
        pltpu.SemaphoreType.DMA((_NCHUNK,)),
        pltpu.SemaphoreType.DMA,
    ],
)


def kernel(X, table, shared_embed):
    x = X.astype(jnp.int32).reshape(_NW, _NCHUNK, _CHUNK)
    sh = shared_embed.reshape(_SHARED)
    return _emb_lookup(x, table, sh)


# speedup vs baseline: 1.0057x; 1.0057x over previous
import functools

import jax
import jax.numpy as jnp
from jax import lax
from jax.experimental import pallas as pl
from jax.experimental.pallas import tpu as pltpu
from jax.experimental.pallas import tpu_sc as plsc

_D = 128
_SHARED = 32
_B = 16384
_NC = 2
_NS = 16
_NW = _NC * _NS
_BPW = _B // _NW
_CHUNK = 64
_NCHUNK = _BPW // _CHUNK


def _sc_body(x_hbm, tab_hbm, sh_hbm, out_hbm, idx_v, rows_v, sh_v, shblk_v, gsems, osem):
    wid = lax.axis_index("s") * _NC + lax.axis_index("c")
    base = wid * _BPW

    a_idx = pltpu.async_copy(x_hbm.at[wid], idx_v, osem)
    a_sh = pltpu.async_copy(sh_hbm, sh_v, osem)
    a_idx.wait()

    gathers = [
        pltpu.async_copy(
            tab_hbm.at[idx_v.at[k]],
            rows_v.at[pl.ds(k * _CHUNK, _CHUNK)],
            gsems.at[k],
        )
        for k in range(_NCHUNK)
    ]
    a_sh.wait()

    s0 = sh_v[pl.ds(0, 16)]
    s1 = sh_v[pl.ds(16, 16)]

    def blk_fill(i, carry):
        shblk_v[i, pl.ds(0, 16)] = s0
        shblk_v[i, pl.ds(16, 16)] = s1
        return carry

    lax.fori_loop(0, _BPW, blk_fill, 0)

    outs = [
        pltpu.async_copy(
            shblk_v,
            out_hbm.at[pl.ds(base, _BPW), pl.ds(0, _SHARED)],
            osem,
        )
    ]
    for k in range(_NCHUNK):
        gathers[k].wait()
        outs.append(
            pltpu.async_copy(
                rows_v.at[pl.ds(k * _CHUNK, _CHUNK), pl.ds(_SHARED, _D - _SHARED)],
                out_hbm.at[pl.ds(base + k * _CHUNK, _CHUNK), pl.ds(_SHARED, _D - _SHARED)],
                osem,
            )
        )
    for o in outs:
        o.wait()


_mesh = plsc.VectorSubcoreMesh(core_axis_name="c", subcore_axis_name="s")

_emb_lookup = pl.kernel(
    _sc_body,
    mesh=_mesh,
    compiler_params=pltpu.CompilerParams(use_tc_tiling_on_sc=False),
    out_type=jax.ShapeDtypeStruct((_B, _D), jnp.float32),
    scratch_types=[
        pltpu.VMEM((_NCHUNK, _CHUNK), jnp.int32),
        pltpu.VMEM((_BPW, _D), jnp.float32),
        pltpu.VMEM((_SHARED,), jnp.float32),
        pltpu.VMEM((_BPW, _SHARED), jnp.float32),
        pltpu.SemaphoreType.DMA((_NCHUNK,)),
        pltpu.SemaphoreType.DMA,
    ],
)


def kernel(X, table, shared_embed):
    x = X.astype(jnp.int32).reshape(_NW, _NCHUNK, _CHUNK)
    sh = shared_embed.reshape(_SHARED)
    return _emb_lookup(x, table, sh)
